# Initial kernel scaffold; baseline (speedup 1.0000x reference)
import jax
import jax.numpy as jnp
from jax.experimental import pallas as pl


def kernel(cost, edge):
    raise NotImplementedError("write your pallas kernel here")


# speedup vs baseline: 31.6774x; 31.6774x over previous
import functools

import jax
import jax.numpy as jnp
from jax.experimental import pallas as pl

_P1 = 0.1
_P2 = 1.0
_INF = 1e9
_DW = 9
_D = 81


def _msg(L, m_dw8, m_dw0):
    P = L.shape[1]
    inf9 = jnp.full((_DW, P), _INF, L.dtype)
    inf1 = jnp.full((1, P), _INF, L.dtype)
    up = jnp.concatenate([L[_DW:], inf9], axis=0)
    down = jnp.concatenate([inf9, L[: _D - _DW]], axis=0)
    lf = jnp.where(m_dw8, _INF, jnp.concatenate([L[1:], inf1], axis=0))
    rt = jnp.where(m_dw0, _INF, jnp.concatenate([inf1, L[:-1]], axis=0))
    nmin = jnp.minimum(jnp.minimum(up, down), jnp.minimum(lf, rt))
    minall = jnp.min(L, axis=0, keepdims=True)
    return jnp.minimum(jnp.minimum(L, nmin + _P1), minall + _P2) - minall


def _scan_kernel(c_ref, e_ref, o_ref, *, T):
    P = c_ref.shape[2]
    d_idx = jax.lax.broadcasted_iota(jnp.int32, (_D, P), 0)
    dw = d_idx % _DW
    m_dw8 = dw == _DW - 1
    m_dw0 = dw == 0

    L0 = c_ref[0]
    o_ref[0] = L0

    def fwd(t, L):
        L = c_ref[t] + e_ref[0, t][None, :] * _msg(L, m_dw8, m_dw0)
        o_ref[t] = L
        return L

    jax.lax.fori_loop(1, T, fwd, L0)

    Lb = c_ref[T - 1]
    o_ref[T - 1] += Lb

    def bwd(i, L):
        t = T - 2 - i
        L = c_ref[t] + e_ref[1, t][None, :] * _msg(L, m_dw8, m_dw0)
        o_ref[t] += L
        return L

    jax.lax.fori_loop(0, T - 1, bwd, Lb)


def _pass_pair(c, e, *, interpret=False):
    T, D, P = c.shape
    return pl.pallas_call(
        functools.partial(_scan_kernel, T=T),
        out_shape=jax.ShapeDtypeStruct((T, D, P), c.dtype),
        interpret=interpret,
    )(c, e)


def kernel(cost, edge, *, interpret=False):
    c = cost[0]
    c_h = jnp.transpose(c, (2, 0, 1))
    c_v = jnp.transpose(c, (1, 0, 2))
    e_h = jnp.transpose(edge[0, 0:2], (0, 2, 1))
    e_v = edge[0, 2:4]
    o_h = _pass_pair(c_h, e_h, interpret=interpret)
    o_v = _pass_pair(c_v, e_v, interpret=interpret)
    out = jnp.transpose(o_h, (1, 2, 0)) + jnp.transpose(o_v, (1, 0, 2))
    return out[None]
